# baseline (speedup 1.0000x reference)
import functools

import jax
import jax.numpy as jnp
from jax import lax
from jax.experimental import pallas as pl
from jax.experimental.pallas import tpu as pltpu
from jax.experimental.pallas import tpu_sc as plsc

_L = 16
_G = 128


def _permute_sc(x_flat, perm_i32, n_rows_sc, n_cols):
    info = plsc.get_sparse_core_info()
    num_cores, num_subcores = info.num_cores, info.num_subcores
    n_workers = num_cores * num_subcores
    rows_per_w = n_rows_sc // n_workers
    chunk_rows = 8
    n_chunks = rows_per_w // chunk_rows
    vecs_per_row = n_cols // _L
    chunk_elems = chunk_rows * n_cols

    mesh = plsc.VectorSubcoreMesh(core_axis_name="c", subcore_axis_name="s")

    @functools.partial(
        pl.kernel,
        out_type=jax.ShapeDtypeStruct((n_rows_sc * n_cols,), jnp.float32),
        mesh=mesh,
        scratch_types=[
            pltpu.VMEM((n_cols,), jnp.int32),
            pltpu.VMEM((chunk_elems,), jnp.float32),
            pltpu.VMEM((chunk_elems,), jnp.float32),
            pltpu.VMEM((chunk_elems,), jnp.float32),
            pltpu.VMEM((chunk_elems,), jnp.float32),
            pltpu.SemaphoreType.DMA,
            pltpu.SemaphoreType.DMA,
            pltpu.SemaphoreType.DMA,
            pltpu.SemaphoreType.DMA,
        ],
        compiler_params=pltpu.CompilerParams(needs_layout_passes=False),
    )
    def k(x_hbm, perm_hbm, out_hbm, perm_v, in0, in1, ot0, ot1, is0, is1, os0, os1):
        wid = lax.axis_index("s") * num_cores + lax.axis_index("c")
        base = wid * rows_per_w * n_cols
        in_bufs = (in0, in1)
        out_bufs = (ot0, ot1)
        in_sems = (is0, is1)
        out_sems = (os0, os1)
        pltpu.sync_copy(perm_hbm, perm_v)

        def start_in(g, b):
            pltpu.async_copy(
                x_hbm.at[pl.ds(base + g * chunk_elems, chunk_elems)], in_bufs[b],
                in_sems[b],
            )

        def wait_in(b):
            pltpu.make_async_copy(
                x_hbm.at[pl.ds(0, chunk_elems)], in_bufs[b], in_sems[b]
            ).wait()

        def start_out(g, b):
            pltpu.async_copy(
                out_bufs[b], out_hbm.at[pl.ds(base + g * chunk_elems, chunk_elems)],
                out_sems[b],
            )

        def wait_out(b):
            pltpu.make_async_copy(
                out_bufs[b], out_hbm.at[pl.ds(0, chunk_elems)], out_sems[b]
            ).wait()

        def compute(b):
            @plsc.parallel_loop(0, n_cols, step=_L, unroll=8)
            def col_body(cbase):
                col = perm_v[pl.ds(cbase, _L)]
                for r in range(chunk_rows):
                    val = plsc.load_gather(in_bufs[b], [col + r * n_cols])
                    out_bufs[b][pl.ds(r * n_cols + cbase, _L)] = val

        start_in(0, 0)
        start_in(1, 1)
        for b in range(2):
            wait_in(b)
            compute(b)
            start_out(b, b)
            start_in(b + 2, b)

        def chunk_body(i, carry):
            g0 = 2 + 2 * i
            for b in range(2):
                g = g0 + b
                wait_in(b)
                wait_out(b)
                compute(b)
                start_out(g, b)

                @pl.when(g + 2 < n_chunks)
                def _():
                    start_in(g + 2, b)

            return carry

        lax.fori_loop(0, (n_chunks - 2) // 2, chunk_body, 0, unroll=1)
        wait_out(0)
        wait_out(1)

    return k(x_flat, perm_i32)


def _permute_tc(x, idxm, idxd, row0, n_rows_tc):
    n_rows, n_cols = x.shape
    n_grp = n_cols // _G
    blk = 32
    blk0 = row0 // blk

    def body(x_ref, idxm_ref, idxd_ref, o_ref):
        xb = x_ref[...]
        for g in range(n_grp):
            im = jnp.broadcast_to(idxm_ref[g:g + 1], (blk, _G))
            imd = jnp.broadcast_to(idxd_ref[g:g + 1], (blk, _G))
            parts = []
            for h in range(n_grp):
                got = jnp.take_along_axis(
                    xb[:, h * _G:(h + 1) * _G], im, axis=1
                )
                parts.append(jnp.where(imd == h, got, 0.0))
            while len(parts) > 1:
                parts = [a + b for a, b in zip(parts[::2], parts[1::2])]
            o_ref[:, g * _G:(g + 1) * _G] = parts[0]

    return pl.pallas_call(
        body,
        grid=(n_rows_tc // blk,),
        in_specs=[
            pl.BlockSpec((blk, n_cols), lambda i: (i + blk0, 0)),
            pl.BlockSpec((n_grp, _G), lambda i: (0, 0)),
            pl.BlockSpec((n_grp, _G), lambda i: (0, 0)),
        ],
        out_specs=pl.BlockSpec((blk, n_cols), lambda i: (i, 0)),
        out_shape=jax.ShapeDtypeStruct((n_rows_tc, n_cols), jnp.float32),
    )(x, idxm, idxd)


def kernel(x, perm):
    n_rows, n_cols = x.shape
    n_grp = n_cols // _G
    perm_i32 = perm.astype(jnp.int32)
    n_sc = 10240
    pg = perm_i32.reshape(n_grp, _G)
    idxm = pg & (_G - 1)
    idxd = pg >> 7
    sc_out = _permute_sc(
        x.reshape(n_rows * n_cols), perm_i32, n_sc, n_cols
    ).reshape(n_sc, n_cols)
    tc_out = _permute_tc(x, idxm, idxd, n_sc, n_rows - n_sc)
    return jnp.concatenate([sc_out, tc_out], axis=0)


# speedup vs baseline: 1.5661x; 1.5661x over previous
import functools

import jax
import jax.numpy as jnp
from jax import lax
from jax.experimental import pallas as pl
from jax.experimental.pallas import tpu as pltpu
from jax.experimental.pallas import tpu_sc as plsc

_L = 16
_NBUF = 4


def _permute_cols_sc(x_flat, perm_i32, n_rows, n_cols):
    info = plsc.get_sparse_core_info()
    num_cores, num_subcores = info.num_cores, info.num_subcores
    n_workers = num_cores * num_subcores
    rows_per_w = n_rows // n_workers
    chunk_rows = 4
    n_chunks = rows_per_w // chunk_rows
    chunk_elems = chunk_rows * n_cols

    mesh = plsc.VectorSubcoreMesh(core_axis_name="c", subcore_axis_name="s")

    @functools.partial(
        pl.kernel,
        out_type=jax.ShapeDtypeStruct((n_rows * n_cols,), jnp.float32),
        mesh=mesh,
        scratch_types=[
            pltpu.VMEM((n_cols,), jnp.int32),
        ]
        + [pltpu.VMEM((chunk_elems,), jnp.float32) for _ in range(2 * _NBUF)]
        + [pltpu.SemaphoreType.DMA for _ in range(2 * _NBUF)],
        compiler_params=pltpu.CompilerParams(needs_layout_passes=False),
    )
    def k(x_hbm, perm_hbm, out_hbm, perm_v, *bufs_and_sems):
        in_bufs = bufs_and_sems[0:_NBUF]
        out_bufs = bufs_and_sems[_NBUF:2 * _NBUF]
        in_sems = bufs_and_sems[2 * _NBUF:3 * _NBUF]
        out_sems = bufs_and_sems[3 * _NBUF:4 * _NBUF]
        wid = lax.axis_index("s") * num_cores + lax.axis_index("c")
        base = wid * rows_per_w * n_cols
        pltpu.sync_copy(perm_hbm, perm_v)

        def start_in(g, b):
            pltpu.async_copy(
                x_hbm.at[pl.ds(base + g * chunk_elems, chunk_elems)], in_bufs[b],
                in_sems[b],
            )

        def wait_in(b):
            pltpu.make_async_copy(
                x_hbm.at[pl.ds(0, chunk_elems)], in_bufs[b], in_sems[b]
            ).wait()

        def start_out(g, b):
            pltpu.async_copy(
                out_bufs[b], out_hbm.at[pl.ds(base + g * chunk_elems, chunk_elems)],
                out_sems[b],
            )

        def wait_out(b):
            pltpu.make_async_copy(
                out_bufs[b], out_hbm.at[pl.ds(0, chunk_elems)], out_sems[b]
            ).wait()

        def compute(b):
            @plsc.parallel_loop(0, n_cols, step=_L, unroll=8)
            def col_body(cbase):
                col = perm_v[pl.ds(cbase, _L)]
                for r in range(chunk_rows):
                    val = plsc.load_gather(in_bufs[b], [col + r * n_cols])
                    out_bufs[b][pl.ds(r * n_cols + cbase, _L)] = val

        for b in range(_NBUF):
            start_in(b, b)
        for g in range(_NBUF):
            wait_in(g)
            compute(g)
            start_out(g, g)
            start_in(g + _NBUF, g)

        def chunk_body(i, carry):
            g0 = _NBUF + _NBUF * i
            for b in range(_NBUF):
                g = g0 + b
                wait_in(b)
                wait_out(b)
                compute(b)
                start_out(g, b)

                @pl.when(g + _NBUF < n_chunks)
                def _():
                    start_in(g + _NBUF, b)

            return carry

        lax.fori_loop(0, (n_chunks - _NBUF) // _NBUF, chunk_body, 0, unroll=1)
        for b in range(_NBUF):
            wait_out(b)

    return k(x_flat, perm_i32)


def kernel(x, perm):
    n_rows, n_cols = x.shape
    out_flat = _permute_cols_sc(
        x.reshape(n_rows * n_cols), perm.astype(jnp.int32), n_rows, n_cols
    )
    return out_flat.reshape(n_rows, n_cols)
